# baseline (speedup 1.0000x reference)
import functools

import jax
import jax.numpy as jnp
from jax import lax
from jax.experimental import pallas as pl
from jax.experimental.pallas import tpu as pltpu
from jax.experimental.pallas import tpu_sc as plsc

N_NODES = 10000
D = 128
E = 320000
SLOPE = 0.01

NC = 2
NS = 16
NW = NC * NS
K = 128
F0 = 40
F1 = 120
CH_MAX = max(F0, F1)
TOT_CH = NS * (F0 + F1)
E_PAD = TOT_CH * K
NP = 10240
ROWS_PER_TILE = NP // NS
DUMMY_DST = N_NODES


def _leaky(v):
    return jnp.where(v >= 0, v, SLOPE * v)


@functools.cache
def _sc_kernels():
    mesh = plsc.VectorSubcoreMesh(
        core_axis_name="c", subcore_axis_name="s",
        num_cores=NC, num_subcores=NS)

    @functools.partial(
        pl.kernel,
        out_type=jax.ShapeDtypeStruct((NC, NP, D), jnp.float32),
        mesh=mesh,
        scratch_types=[
            pltpu.VMEM((CH_MAX, K), jnp.int32),
            pltpu.VMEM((2, K), jnp.int32),
            pltpu.VMEM((2, K), jnp.int32),
            pltpu.VMEM((K, D), jnp.float32),
            pltpu.VMEM((K, D), jnp.float32),
            pltpu.VMEM_SHARED((NP, D), jnp.float32),
            pltpu.SemaphoreType.DMA,
            pltpu.SemaphoreType.DMA,
            pltpu.SemaphoreType.DMA,
            pltpu.SemaphoreType.DMA,
        ],
    )
    def sc_aggregate(h_hbm, packed_hbm, out_hbm,
                     packed_v, src_st, dst_st, rows_a, rows_b, agg_sh,
                     sem_ga, sem_gb, sem_sa, sem_sb):
        c = lax.axis_index("c")
        s = lax.axis_index("s")

        zeros16 = jnp.zeros((16,), jnp.float32)

        def _zrow(i, _):
            for kk in range(D // 16):
                rows_a[i, pl.ds(kk * 16, 16)] = zeros16
            return 0
        lax.fori_loop(0, K, _zrow, 0)
        for t in range(ROWS_PER_TILE // K):
            pltpu.sync_copy(rows_a,
                            agg_sh.at[pl.ds(s * ROWS_PER_TILE + t * K, K)])
        plsc.subcore_barrier()

        def _unpack(j, slot):
            for kk in range(K // 16):
                v = packed_v[j, pl.ds(kk * 16, 16)]
                src_st[slot, pl.ds(kk * 16, 16)] = v & 0xFFFF
                dst_st[slot, pl.ds(kk * 16, 16)] = lax.shift_right_logical(
                    v, 16)

        def _g_start(buf, slot, sem):
            pltpu.async_copy(h_hbm.at[src_st.at[slot]], buf, sem)

        def _g_wait(buf, slot, sem):
            pltpu.make_async_copy(h_hbm.at[src_st.at[slot]], buf, sem).wait()

        def _s_start(buf, slot, sem):
            pltpu.async_copy(buf, agg_sh.at[dst_st.at[slot]], sem, add=True)

        def _s_wait(buf, slot, sem):
            pltpu.make_async_copy(buf, agg_sh.at[dst_st.at[slot]], sem).wait()

        def _run(base, n_ch):
            pltpu.sync_copy(packed_hbm.at[pl.ds(base, n_ch)],
                            packed_v.at[pl.ds(0, n_ch)])
            _unpack(0, 0)
            _g_start(rows_a, 0, sem_ga)
            _unpack(1, 1)
            _g_start(rows_b, 1, sem_gb)

            def _pair(t, _):
                ja = 2 * t
                jb = 2 * t + 1
                _g_wait(rows_a, 0, sem_ga)
                _s_start(rows_a, 0, sem_sa)
                _g_wait(rows_b, 1, sem_gb)
                _s_start(rows_b, 1, sem_sb)
                _s_wait(rows_a, 0, sem_sa)
                @pl.when(ja + 2 < n_ch)
                def _():
                    _unpack(ja + 2, 0)
                    _g_start(rows_a, 0, sem_ga)
                _s_wait(rows_b, 1, sem_sb)
                @pl.when(jb + 2 < n_ch)
                def _():
                    _unpack(jb + 2, 1)
                    _g_start(rows_b, 1, sem_gb)
                return 0

            lax.fori_loop(0, n_ch // 2, _pair, 0)

        @pl.when(c == 0)
        def _():
            _run(s * F0, F0)

        @pl.when(c == 1)
        def _():
            _run(NS * F0 + s * F1, F1)

        plsc.subcore_barrier()

        pltpu.sync_copy(agg_sh.at[pl.ds(s * ROWS_PER_TILE, ROWS_PER_TILE)],
                        out_hbm.at[c, pl.ds(s * ROWS_PER_TILE, ROWS_PER_TILE)])

    @functools.partial(
        pl.kernel,
        out_type=jax.ShapeDtypeStruct((NC, NP), jnp.float32),
        mesh=mesh,
        scratch_types=[
            pltpu.VMEM((TOT_CH // NW, K), jnp.int32),
            pltpu.VMEM((1, K), jnp.int32),
            pltpu.VMEM((K,), jnp.float32),
            pltpu.VMEM_SHARED((NP,), jnp.float32),
        ],
    )
    def sc_degree(packed_hbm, out_hbm, packed_v, dst_st, ones_v, deg_sh):
        c = lax.axis_index("c")
        s = lax.axis_index("s")
        w = c * NS + s
        ch_deg = TOT_CH // NW

        pltpu.sync_copy(packed_hbm.at[pl.ds(w * ch_deg, ch_deg)], packed_v)
        zero16 = jnp.zeros((16,), jnp.float32)
        one16 = jnp.full((16,), 1.0, jnp.float32)

        for kk in range(K // 16):
            ones_v[pl.ds(kk * 16, 16)] = zero16
        for t in range(ROWS_PER_TILE // K):
            pltpu.sync_copy(ones_v,
                            deg_sh.at[pl.ds(s * ROWS_PER_TILE + t * K, K)])
        for kk in range(K // 16):
            ones_v[pl.ds(kk * 16, 16)] = one16
        plsc.subcore_barrier()

        def _chunk(j, _):
            for kk in range(K // 16):
                v = packed_v[j, pl.ds(kk * 16, 16)]
                dst_st[0, pl.ds(kk * 16, 16)] = lax.shift_right_logical(v, 16)
            pltpu.sync_copy(ones_v, deg_sh.at[dst_st.at[0]], add=True)
            return 0
        lax.fori_loop(0, ch_deg, _chunk, 0)
        plsc.subcore_barrier()

        pltpu.sync_copy(deg_sh.at[pl.ds(s * ROWS_PER_TILE, ROWS_PER_TILE)],
                        out_hbm.at[c, pl.ds(s * ROWS_PER_TILE, ROWS_PER_TILE)])

    return sc_aggregate, sc_degree


_RB = 1024
_GRID = NP // _RB


def _fc1_body(x_ref, w_ref, b_ref, o_ref):
    o_ref[...] = _leaky(
        jnp.dot(x_ref[...], w_ref[...], preferred_element_type=jnp.float32)
        + b_ref[...])


def _tc_fc1(x, W1, b1):
    return pl.pallas_call(
        _fc1_body,
        grid=(_GRID,),
        in_specs=[
            pl.BlockSpec((_RB, D), lambda i: (i, 0)),
            pl.BlockSpec((D, D), lambda i: (0, 0)),
            pl.BlockSpec((1, D), lambda i: (0, 0)),
        ],
        out_specs=pl.BlockSpec((_RB, D), lambda i: (i, 0)),
        out_shape=jax.ShapeDtypeStruct((NP, D), jnp.float32),
    )(x, W1, b1)


def _combine_body(a_ref, deg_ref, h_ref, w_ref, b_ref, o_ref):
    agg = a_ref[0] + a_ref[1]
    deg = deg_ref[0] + deg_ref[1]
    deg = deg.reshape(_RB, 1)
    agg = jnp.where(deg > 0, agg / jnp.maximum(deg, 1.0), 0.0)
    o_ref[...] = _leaky(
        jnp.dot(agg, w_ref[...], preferred_element_type=jnp.float32)
        + b_ref[...] + h_ref[...])


def _tc_combine(aggpair, degpair, h, W2, b2):
    return pl.pallas_call(
        _combine_body,
        grid=(_GRID,),
        in_specs=[
            pl.BlockSpec((NC, _RB, D), lambda i: (0, i, 0)),
            pl.BlockSpec((NC, _RB), lambda i: (0, i)),
            pl.BlockSpec((_RB, D), lambda i: (i, 0)),
            pl.BlockSpec((D, D), lambda i: (0, 0)),
            pl.BlockSpec((1, D), lambda i: (0, 0)),
        ],
        out_specs=pl.BlockSpec((_RB, D), lambda i: (i, 0)),
        out_shape=jax.ShapeDtypeStruct((NP, D), jnp.float32),
    )(aggpair, degpair, h, W2, b2)


def kernel(x, edge_index, W1, b1, W2, b2, num_iterations):
    src = edge_index[0].astype(jnp.int32)
    dst = edge_index[1].astype(jnp.int32)
    pad = E_PAD - E
    src_p = jnp.concatenate([src, jnp.zeros((pad,), jnp.int32)])
    dst_p = jnp.concatenate([dst, jnp.full((pad,), DUMMY_DST, jnp.int32)])
    packed = ((dst_p << 16) | src_p).reshape(TOT_CH, K)

    x_pad = jnp.zeros((NP, D), jnp.float32).at[:N_NODES].set(x)
    b1_2d = b1.reshape(1, D)
    b2_2d = b2.reshape(1, D)

    sc_aggregate, sc_degree = _sc_kernels()
    degpair = sc_degree(packed)
    h = _tc_fc1(x_pad, W1, b1_2d)

    def _body(_, h):
        aggpair = sc_aggregate(h, packed)
        return _tc_combine(aggpair, degpair, h, W2, b2_2d)

    h = lax.fori_loop(0, num_iterations, _body, h)
    return h[:N_NODES]


# speedup vs baseline: 1.1984x; 1.1984x over previous
import functools

import jax
import jax.numpy as jnp
from jax import lax
from jax.experimental import pallas as pl
from jax.experimental.pallas import tpu as pltpu
from jax.experimental.pallas import tpu_sc as plsc

N_NODES = 10000
D = 128
E = 320000
SLOPE = 0.01

NC = 2
NS = 16
NW = NC * NS
K = 128
F0 = 120
F1 = 40
CH_MAX = max(F0, F1)
TOT_CH = NS * (F0 + F1)
E_PAD = TOT_CH * K
NP = 10240
ROWS_PER_TILE = NP // NS
DUMMY_DST = N_NODES


def _leaky(v):
    return jnp.where(v >= 0, v, SLOPE * v)


@functools.cache
def _sc_kernels():
    mesh = plsc.VectorSubcoreMesh(
        core_axis_name="c", subcore_axis_name="s",
        num_cores=NC, num_subcores=NS)

    @functools.partial(
        pl.kernel,
        out_type=jax.ShapeDtypeStruct((NC, NP, D), jnp.float32),
        mesh=mesh,
        scratch_types=[
            pltpu.VMEM((CH_MAX, K), jnp.int32),
            pltpu.VMEM((2, K), jnp.int32),
            pltpu.VMEM((2, K), jnp.int32),
            pltpu.VMEM((K, D), jnp.float32),
            pltpu.VMEM((K, D), jnp.float32),
            pltpu.VMEM_SHARED((NP, D), jnp.float32),
            pltpu.SemaphoreType.DMA,
            pltpu.SemaphoreType.DMA,
            pltpu.SemaphoreType.DMA,
            pltpu.SemaphoreType.DMA,
        ],
    )
    def sc_aggregate(h_hbm, packed_hbm, out_hbm,
                     packed_v, src_st, dst_st, rows_a, rows_b, agg_sh,
                     sem_ga, sem_gb, sem_sa, sem_sb):
        c = lax.axis_index("c")
        s = lax.axis_index("s")

        zeros16 = jnp.zeros((16,), jnp.float32)

        def _zrow(i, _):
            for kk in range(D // 16):
                rows_a[i, pl.ds(kk * 16, 16)] = zeros16
            return 0
        lax.fori_loop(0, K, _zrow, 0)
        for t in range(ROWS_PER_TILE // K):
            pltpu.sync_copy(rows_a,
                            agg_sh.at[pl.ds(s * ROWS_PER_TILE + t * K, K)])
        plsc.subcore_barrier()

        def _unpack(j, slot):
            for kk in range(K // 16):
                v = packed_v[j, pl.ds(kk * 16, 16)]
                src_st[slot, pl.ds(kk * 16, 16)] = v & 0xFFFF
                dst_st[slot, pl.ds(kk * 16, 16)] = lax.shift_right_logical(
                    v, 16)

        def _g_start(buf, slot, sem):
            pltpu.async_copy(h_hbm.at[src_st.at[slot]], buf, sem)

        def _g_wait(buf, slot, sem):
            pltpu.make_async_copy(h_hbm.at[src_st.at[slot]], buf, sem).wait()

        def _s_start(buf, slot, sem):
            pltpu.async_copy(buf, agg_sh.at[dst_st.at[slot]], sem, add=True)

        def _s_wait(buf, slot, sem):
            pltpu.make_async_copy(buf, agg_sh.at[dst_st.at[slot]], sem).wait()

        def _run(base, n_ch):
            pltpu.sync_copy(packed_hbm.at[pl.ds(base, n_ch)],
                            packed_v.at[pl.ds(0, n_ch)])
            _unpack(0, 0)
            _g_start(rows_a, 0, sem_ga)
            _unpack(1, 1)
            _g_start(rows_b, 1, sem_gb)

            def _pair(t, _):
                ja = 2 * t
                jb = 2 * t + 1
                _g_wait(rows_a, 0, sem_ga)
                _s_start(rows_a, 0, sem_sa)
                _g_wait(rows_b, 1, sem_gb)
                _s_start(rows_b, 1, sem_sb)
                _s_wait(rows_a, 0, sem_sa)
                @pl.when(ja + 2 < n_ch)
                def _():
                    _unpack(ja + 2, 0)
                    _g_start(rows_a, 0, sem_ga)
                _s_wait(rows_b, 1, sem_sb)
                @pl.when(jb + 2 < n_ch)
                def _():
                    _unpack(jb + 2, 1)
                    _g_start(rows_b, 1, sem_gb)
                return 0

            lax.fori_loop(0, n_ch // 2, _pair, 0)

        @pl.when(c == 0)
        def _():
            _run(s * F0, F0)

        @pl.when(c == 1)
        def _():
            _run(NS * F0 + s * F1, F1)

        plsc.subcore_barrier()

        pltpu.sync_copy(agg_sh.at[pl.ds(s * ROWS_PER_TILE, ROWS_PER_TILE)],
                        out_hbm.at[c, pl.ds(s * ROWS_PER_TILE, ROWS_PER_TILE)])

    @functools.partial(
        pl.kernel,
        out_type=jax.ShapeDtypeStruct((NC, NP), jnp.float32),
        mesh=mesh,
        scratch_types=[
            pltpu.VMEM((TOT_CH // NW, K), jnp.int32),
            pltpu.VMEM((1, K), jnp.int32),
            pltpu.VMEM((K,), jnp.float32),
            pltpu.VMEM_SHARED((NP,), jnp.float32),
        ],
    )
    def sc_degree(packed_hbm, out_hbm, packed_v, dst_st, ones_v, deg_sh):
        c = lax.axis_index("c")
        s = lax.axis_index("s")
        w = c * NS + s
        ch_deg = TOT_CH // NW

        pltpu.sync_copy(packed_hbm.at[pl.ds(w * ch_deg, ch_deg)], packed_v)
        zero16 = jnp.zeros((16,), jnp.float32)
        one16 = jnp.full((16,), 1.0, jnp.float32)

        for kk in range(K // 16):
            ones_v[pl.ds(kk * 16, 16)] = zero16
        for t in range(ROWS_PER_TILE // K):
            pltpu.sync_copy(ones_v,
                            deg_sh.at[pl.ds(s * ROWS_PER_TILE + t * K, K)])
        for kk in range(K // 16):
            ones_v[pl.ds(kk * 16, 16)] = one16
        plsc.subcore_barrier()

        def _chunk(j, _):
            for kk in range(K // 16):
                v = packed_v[j, pl.ds(kk * 16, 16)]
                dst_st[0, pl.ds(kk * 16, 16)] = lax.shift_right_logical(v, 16)
            pltpu.sync_copy(ones_v, deg_sh.at[dst_st.at[0]], add=True)
            return 0
        lax.fori_loop(0, ch_deg, _chunk, 0)
        plsc.subcore_barrier()

        pltpu.sync_copy(deg_sh.at[pl.ds(s * ROWS_PER_TILE, ROWS_PER_TILE)],
                        out_hbm.at[c, pl.ds(s * ROWS_PER_TILE, ROWS_PER_TILE)])

    return sc_aggregate, sc_degree


_RB = 1024
_GRID = NP // _RB


def _fc1_body(x_ref, w_ref, b_ref, o_ref):
    o_ref[...] = _leaky(
        jnp.dot(x_ref[...], w_ref[...], preferred_element_type=jnp.float32)
        + b_ref[...])


def _tc_fc1(x, W1, b1):
    return pl.pallas_call(
        _fc1_body,
        grid=(_GRID,),
        in_specs=[
            pl.BlockSpec((_RB, D), lambda i: (i, 0)),
            pl.BlockSpec((D, D), lambda i: (0, 0)),
            pl.BlockSpec((1, D), lambda i: (0, 0)),
        ],
        out_specs=pl.BlockSpec((_RB, D), lambda i: (i, 0)),
        out_shape=jax.ShapeDtypeStruct((NP, D), jnp.float32),
    )(x, W1, b1)


def _combine_body(a_ref, deg_ref, h_ref, w_ref, b_ref, o_ref):
    agg = a_ref[0] + a_ref[1]
    deg = deg_ref[0] + deg_ref[1]
    deg = deg.reshape(_RB, 1)
    agg = jnp.where(deg > 0, agg / jnp.maximum(deg, 1.0), 0.0)
    o_ref[...] = _leaky(
        jnp.dot(agg, w_ref[...], preferred_element_type=jnp.float32)
        + b_ref[...] + h_ref[...])


def _tc_combine(aggpair, degpair, h, W2, b2):
    return pl.pallas_call(
        _combine_body,
        grid=(_GRID,),
        in_specs=[
            pl.BlockSpec((NC, _RB, D), lambda i: (0, i, 0)),
            pl.BlockSpec((NC, _RB), lambda i: (0, i)),
            pl.BlockSpec((_RB, D), lambda i: (i, 0)),
            pl.BlockSpec((D, D), lambda i: (0, 0)),
            pl.BlockSpec((1, D), lambda i: (0, 0)),
        ],
        out_specs=pl.BlockSpec((_RB, D), lambda i: (i, 0)),
        out_shape=jax.ShapeDtypeStruct((NP, D), jnp.float32),
    )(aggpair, degpair, h, W2, b2)


def kernel(x, edge_index, W1, b1, W2, b2, num_iterations):
    src = edge_index[0].astype(jnp.int32)
    dst = edge_index[1].astype(jnp.int32)
    pad = E_PAD - E
    src_p = jnp.concatenate([src, jnp.zeros((pad,), jnp.int32)])
    dst_p = jnp.concatenate([dst, jnp.full((pad,), DUMMY_DST, jnp.int32)])
    packed = ((dst_p << 16) | src_p).reshape(TOT_CH, K)

    x_pad = jnp.zeros((NP, D), jnp.float32).at[:N_NODES].set(x)
    b1_2d = b1.reshape(1, D)
    b2_2d = b2.reshape(1, D)

    sc_aggregate, sc_degree = _sc_kernels()
    degpair = sc_degree(packed)
    h = _tc_fc1(x_pad, W1, b1_2d)

    def _body(_, h):
        aggpair = sc_aggregate(h, packed)
        return _tc_combine(aggpair, degpair, h, W2, b2_2d)

    h = lax.fori_loop(0, num_iterations, _body, h)
    return h[:N_NODES]
